# baseline (speedup 1.0000x reference)
import functools

import jax
import jax.numpy as jnp
from jax import lax
from jax.experimental import pallas as pl
from jax.experimental.pallas import tpu as pltpu
from jax.experimental.pallas import tpu_sc as plsc

NC, NS, LANES = 2, 16, 16
NW = NC * NS
EMB = 64
SEQ = 50
K = 4 * EMB
V3 = 1000
VPAD = 1024
RPW = 32
JPT = 13

_mesh = plsc.VectorSubcoreMesh(core_axis_name="c", subcore_axis_name="s",
                               num_cores=NC, num_subcores=NS)
_sc_params = pltpu.CompilerParams(needs_layout_passes=False,
                                  use_tc_tiling_on_sc=True)


def _splat2(ref2, j, ncols):
    r = jnp.broadcast_to(j // ncols, (LANES,)).astype(jnp.int32)
    q = jnp.broadcast_to(j % ncols, (LANES,)).astype(jnp.int32)
    return plsc.load_gather(ref2, [r, q])


def _splat(ref, j):
    return plsc.load_gather(ref, [jnp.broadcast_to(j, (LANES,)).astype(jnp.int32)])


@functools.partial(
    pl.kernel,
    out_type=(jax.ShapeDtypeStruct((NC, 4, 1, 128), jnp.float32),
              jax.ShapeDtypeStruct((NC, 4, 1, 128), jnp.float32)),
    mesh=_mesh,
    compiler_params=_sc_params,
    scratch_types=[
        pltpu.VMEM((SEQ + 22,), jnp.int32),
        pltpu.VMEM((JPT, 8, EMB), jnp.float32),
        pltpu.VMEM((4, LANES), jnp.float32),
        pltpu.VMEM((EMB,), jnp.float32),
        pltpu.VMEM((2, 128), jnp.float32),
        pltpu.VMEM((16, 2, 128), jnp.float32),
        pltpu.VMEM((2, 128), jnp.float32),
        pltpu.VMEM((K, 128), jnp.float32),
        pltpu.VMEM((128,), jnp.float32),
        pltpu.VMEM((128,), jnp.float32),
        pltpu.VMEM((16, 128), jnp.float32),
        pltpu.VMEM((1, 128), jnp.float32),
        pltpu.VMEM((1, 128), jnp.float32),
        pltpu.VMEM_SHARED((16, 2, 128), jnp.float32),
        pltpu.VMEM_SHARED((16, 128), jnp.float32),
        pltpu.SemaphoreType.DMA,
    ],
)
def _fwd_kernel(d_i, p_i, s_i, m_i, e0, e1, e2, e3, fuse_h, w_h, b_h,
                res_o, p_o,
                idx_v, blk_v, fuse_v, ps_v, cont_v, reps_v, rep_v,
                w_v, b_v, lp_v, lps_v, rrow_v, prow_v,
                shrep, shlin, sem):
    c = lax.axis_index("c")
    s = lax.axis_index("s")
    zero = jnp.zeros((LANES,), jnp.float32)
    pltpu.sync_copy(fuse_h, fuse_v)

    lane4 = s % 4
    idx_srcs = (d_i, p_i, s_i, m_i)
    tabs = (e0, e1, e2, e3)
    for tt in range(4):
        @pl.when(s // 4 == tt)
        def _(tt=tt):
            pltpu.sync_copy(idx_srcs[tt], idx_v.at[pl.ds(0, SEQ)])
            for cc in range(4):
                ps_v[pl.ds(16 * cc, LANES)] = zero
            for m in range(JPT):
                j = lane4 * JPT + m

                @pl.when(j < SEQ)
                def _(m=m, j=j):
                    a = idx_v[pl.ds(j, 16)][0]
                    blk = pl.multiple_of((a // 8) * 8, 8)
                    pltpu.async_copy(tabs[tt].at[pl.ds(blk, 8), :],
                                     blk_v.at[m], sem)
            for m in range(JPT):
                j = lane4 * JPT + m

                @pl.when(j < SEQ)
                def _(m=m, j=j):
                    pltpu.make_async_copy(tabs[tt].at[pl.ds(0, 8), :],
                                          blk_v.at[m], sem).wait()
            for m in range(JPT):
                j = lane4 * JPT + m

                @pl.when(j < SEQ)
                def _(m=m, j=j):
                    a = idx_v[pl.ds(j, 16)][0]
                    sub = a - (a // 8) * 8
                    for cc in range(4):
                        sl = pl.ds(16 * cc, LANES)
                        ps_v[sl] = ps_v[sl] + blk_v[m, sub, sl]
            ft = fuse_v[tt]
            rr, base = tt // 2, 64 * (tt % 2)
            for ch in range(2):
                for k in range(8):
                    cont_v[ch, pl.ds(16 * k, LANES)] = zero
            for cc in range(4):
                cont_v[rr, pl.ds(base + 16 * cc, LANES)] = \
                    ps_v[pl.ds(16 * cc, LANES)] * ft
    pltpu.sync_copy(cont_v, shrep.at[s])
    plsc.subcore_barrier()

    pltpu.sync_copy(shrep, reps_v)
    for ch in range(2):
        for k in range(8):
            sl = pl.ds(16 * k, LANES)
            acc = zero
            for u in range(16):
                acc = acc + reps_v[u, ch, sl]
            rep_v[ch, sl] = acc

    blkc = 4 * c + s // 4
    q = s % 4
    off = pl.multiple_of(blkc * 128, 128)
    pltpu.sync_copy(w_h.at[:, pl.ds(off, 128)], w_v)
    pltpu.sync_copy(b_h.at[pl.ds(off, 128)], b_v)
    bm = (q == 0).astype(jnp.float32)

    def mv(i, accs):
        jj = q * 64 + i
        sp = _splat2(rep_v, jj, 128)
        return tuple(a + sp * w_v[jj, pl.ds(16 * k, LANES)]
                     for k, a in enumerate(accs))

    accs = lax.fori_loop(0, 64, mv,
                         tuple(b_v[pl.ds(16 * k, LANES)] * bm for k in range(8)))
    for k in range(8):
        lp_v[pl.ds(16 * k, LANES)] = accs[k]
    pltpu.sync_copy(lp_v, shlin.at[s])
    plsc.subcore_barrier()

    @pl.when(s < 4)
    def _():
        pltpu.sync_copy(shlin, lps_v)
        for k in range(8):
            sl = pl.ds(16 * k, LANES)
            acc = zero
            for qq in range(4):
                acc = acc + lps_v[4 * s + qq, sl]
            rrow_v[0, sl] = acc
            prow_v[0, sl] = 1.0 / (1.0 + jnp.exp(-acc))
        pltpu.sync_copy(rrow_v, res_o.at[c, s])
        pltpu.sync_copy(prow_v, p_o.at[c, s])


_NFC = 62


@functools.partial(
    pl.kernel,
    out_type=jax.ShapeDtypeStruct((NW, 1, 128), jnp.float32),
    mesh=_mesh,
    compiler_params=_sc_params,
    scratch_types=[
        pltpu.VMEM((1, V3), jnp.float32),
        pltpu.VMEM((RPW, V3), jnp.float32),
        pltpu.VMEM((1, 128), jnp.float32),
    ],
)
def _ddi_kernel(p_h, a_h, out_h, p_v, a_v, tot_v):
    wid = lax.axis_index("s") * NC + lax.axis_index("c")
    r0 = jnp.minimum(wid * RPW, V3 - RPW)
    zero = jnp.zeros((LANES,), jnp.float32)
    pltpu.sync_copy(p_h, p_v)
    pltpu.sync_copy(a_h.at[pl.ds(r0, RPW), :], a_v)
    tailm = (lax.iota(jnp.int32, LANES) >= 8).astype(jnp.float32)

    def row(r, total):
        def dot(k, acc):
            return acc + a_v[r, pl.ds(16 * k, LANES)] * p_v[0, pl.ds(16 * k, LANES)]

        rowacc = lax.fori_loop(0, _NFC, dot, zero)
        rowacc = rowacc + (a_v[r, pl.ds(984, LANES)] * tailm
                           * p_v[0, pl.ds(984, LANES)])
        g = r0 + r
        coef = _splat(p_v.at[0], g)
        valid = (jnp.broadcast_to(g, (LANES,)) >= wid * RPW).astype(jnp.float32)
        return total + coef * valid * rowacc

    for m in range(1, 8):
        tot_v[0, pl.ds(16 * m, LANES)] = zero
    tot_v[0, pl.ds(0, LANES)] = lax.fori_loop(0, RPW, row, zero)
    pltpu.sync_copy(tot_v, out_h.at[wid])


def kernel(diag_idx, proc_idx, sym_idx, med_idx, emb0, emb1, emb2, emb3,
           fuse_w, W_q, b_q, ddi_adj):
    idx = [i.astype(jnp.int32) for i in (diag_idx, proc_idx, sym_idx, med_idx)]
    fuse16 = jnp.broadcast_to(fuse_w.reshape(4, 1), (4, LANES))
    w_pad = jnp.pad(W_q, ((0, 0), (0, VPAD - V3)))
    b_pad = jnp.pad(b_q, (0, VPAD - V3))
    res4, p4 = _fwd_kernel(*idx, emb0, emb1, emb2, emb3, fuse16, w_pad, b_pad)
    res = res4.reshape(1, VPAD)[:, :V3]
    p = p4.reshape(1, VPAD)[:, :V3]
    partials = _ddi_kernel(p, ddi_adj)
    return res, 0.0005 * jnp.sum(partials)


# speedup vs baseline: 3.3761x; 3.3761x over previous
import functools

import jax
import jax.numpy as jnp
from jax import lax
from jax.experimental import pallas as pl
from jax.experimental.pallas import tpu as pltpu
from jax.experimental.pallas import tpu_sc as plsc

NC, NS, LANES = 2, 16, 16
NW = NC * NS
EMB = 64
SEQ = 50
V3 = 1000
RPW = 32
VOCABS = (100000, 100000, 100000, 1000)
THRS = tuple((v // 128) * 128 for v in VOCABS)
TWS = tuple(v - t for v, t in zip(VOCABS, THRS))

_mesh = plsc.VectorSubcoreMesh(core_axis_name="c", subcore_axis_name="s",
                               num_cores=NC, num_subcores=NS)
_sc_params = pltpu.CompilerParams(needs_layout_passes=False,
                                  use_tc_tiling_on_sc=True)


def _splat(ref, j):
    return plsc.load_gather(ref, [jnp.broadcast_to(j, (LANES,)).astype(jnp.int32)])


def _tc_body(d_i, p_i, s_i, m_i, e0, e1, e2, e3, t0, t1, t2, t3,
             fuse_r, w_r, b_r, res_o, p_o, win_v, sem):
    idxs = (d_i, p_i, s_i, m_i)
    tabs = (e0, e1, e2, e3)
    tails = (t0, t1, t2, t3)
    for t in range(4):
        for j in range(SEQ):
            a = idxs[t][j]
            wb = pl.multiple_of(
                jnp.minimum((a // 128) * 128, THRS[t] - 128), 128)
            pltpu.make_async_copy(
                tabs[t].at[:, pl.ds(wb, 128)],
                win_v.at[t * SEQ + j], sem).start()
    for n in range(4 * SEQ):
        pltpu.make_async_copy(
            tabs[0].at[:, pl.ds(0, 128)], win_v.at[0], sem).wait()

    w = w_r[...]
    res = b_r[...]
    lane = lax.broadcasted_iota(jnp.int32, (EMB, 128), 1)
    for t in range(4):
        acc = jnp.zeros((EMB, 128), jnp.float32)
        tlane = lax.broadcasted_iota(jnp.int32, (EMB, TWS[t]), 1)
        tacc = jnp.zeros((EMB, TWS[t]), jnp.float32)
        tail = tails[t][...]
        for j in range(SEQ):
            a = idxs[t][j]
            wb = jnp.minimum((a // 128) * 128, THRS[t] - 128)
            acc = acc + jnp.where(lane == a - wb, win_v[t * SEQ + j], 0.0)
            tacc = tacc + jnp.where(tlane == a - THRS[t], tail, 0.0)
        pooled = (acc.sum(axis=1, keepdims=True)
                  + tacc.sum(axis=1, keepdims=True)) * fuse_r[t]
        res = res + jnp.dot(w[:, t * EMB:(t + 1) * EMB], pooled,
                            preferred_element_type=jnp.float32)
    res_o[...] = res
    p_o[...] = 1.0 / (1.0 + jnp.exp(-res))


_tc_fwd = pl.pallas_call(
    _tc_body,
    out_shape=(jax.ShapeDtypeStruct((V3, 1), jnp.float32),
               jax.ShapeDtypeStruct((V3, 1), jnp.float32)),
    in_specs=[pl.BlockSpec(memory_space=pltpu.SMEM)] * 4
    + [pl.BlockSpec(memory_space=pl.ANY)] * 4
    + [pl.BlockSpec(memory_space=pltpu.VMEM)] * 4
    + [pl.BlockSpec(memory_space=pltpu.SMEM)]
    + [pl.BlockSpec(memory_space=pltpu.VMEM)] * 2,
    out_specs=(pl.BlockSpec(memory_space=pltpu.VMEM),
               pl.BlockSpec(memory_space=pltpu.VMEM)),
    scratch_shapes=[pltpu.VMEM((4 * SEQ, EMB, 128), jnp.float32),
                    pltpu.SemaphoreType.DMA],
    compiler_params=pltpu.CompilerParams(
        vmem_limit_bytes=48 * 1024 * 1024),
)


_NFC = 62


@functools.partial(
    pl.kernel,
    out_type=jax.ShapeDtypeStruct((NW, 1, 128), jnp.float32),
    mesh=_mesh,
    compiler_params=_sc_params,
    scratch_types=[
        pltpu.VMEM((1, V3), jnp.float32),
        pltpu.VMEM((RPW, V3), jnp.float32),
        pltpu.VMEM((1, 128), jnp.float32),
    ],
)
def _ddi_kernel(p_h, a_h, out_h, p_v, a_v, tot_v):
    wid = lax.axis_index("s") * NC + lax.axis_index("c")
    r0 = jnp.minimum(wid * RPW, V3 - RPW)
    zero = jnp.zeros((LANES,), jnp.float32)
    pltpu.sync_copy(p_h, p_v)
    pltpu.sync_copy(a_h.at[pl.ds(r0, RPW), :], a_v)
    tailm = (lax.iota(jnp.int32, LANES) >= 8).astype(jnp.float32)

    def row(r, total):
        def dot(k, acc):
            return acc + a_v[r, pl.ds(16 * k, LANES)] * p_v[0, pl.ds(16 * k, LANES)]

        rowacc = lax.fori_loop(0, _NFC, dot, zero)
        rowacc = rowacc + (a_v[r, pl.ds(984, LANES)] * tailm
                           * p_v[0, pl.ds(984, LANES)])
        g = r0 + r
        coef = _splat(p_v.at[0], g)
        valid = (jnp.broadcast_to(g, (LANES,)) >= wid * RPW).astype(jnp.float32)
        return total + coef * valid * rowacc

    for m in range(1, 8):
        tot_v[0, pl.ds(16 * m, LANES)] = zero
    tot_v[0, pl.ds(0, LANES)] = lax.fori_loop(0, RPW, row, zero)
    pltpu.sync_copy(tot_v, out_h.at[wid])


def kernel(diag_idx, proc_idx, sym_idx, med_idx, emb0, emb1, emb2, emb3,
           fuse_w, W_q, b_q, ddi_adj):
    idx = [i.astype(jnp.int32) for i in (diag_idx, proc_idx, sym_idx, med_idx)]
    embs = (emb0, emb1, emb2, emb3)
    embsT = [e.T for e in embs]
    tails = [e[t:].T for e, t in zip(embs, THRS)]
    res_c, p_c = _tc_fwd(*idx, *embsT, *tails, fuse_w.reshape(4),
                         W_q.T, b_q[:, None])
    res = res_c.reshape(1, V3)
    p = p_c.reshape(1, V3)
    partials = _ddi_kernel(p, ddi_adj)
    return res, 0.0005 * jnp.sum(partials)


# speedup vs baseline: 3.5535x; 1.0526x over previous
import functools

import jax
import jax.numpy as jnp
from jax import lax
from jax.experimental import pallas as pl
from jax.experimental.pallas import tpu as pltpu
from jax.experimental.pallas import tpu_sc as plsc

NC, NS, LANES = 2, 16, 16
NW = NC * NS
EMB = 64
SEQ = 50
V3 = 1000
RPW = 32
VOCABS = (100000, 100000, 100000, 1000)
THRS = tuple((v // 128) * 128 for v in VOCABS)
TWS = tuple(v - t for v, t in zip(VOCABS, THRS))

_mesh = plsc.VectorSubcoreMesh(core_axis_name="c", subcore_axis_name="s",
                               num_cores=NC, num_subcores=NS)
_sc_params = pltpu.CompilerParams(needs_layout_passes=False,
                                  use_tc_tiling_on_sc=True)


def _splat(ref, j):
    return plsc.load_gather(ref, [jnp.broadcast_to(j, (LANES,)).astype(jnp.int32)])


_TOFF = (0, 32, 64, 96, 200)


def _tc_body(idx_r, e0, e1, e2, e3, tails_r,
             fuse_r, w_r, b_r, res_o, p_o, win_v, sem):
    tabs = (e0, e1, e2, e3)
    for t in range(4):
        for j in range(SEQ):
            a = idx_r[t, j]
            wb = pl.multiple_of(
                jnp.minimum((a // 128) * 128, THRS[t] - 128), 128)
            pltpu.make_async_copy(
                tabs[t].at[:, pl.ds(wb, 128)],
                win_v.at[t * SEQ + j], sem).start()
    for n in range(4 * SEQ):
        pltpu.make_async_copy(
            tabs[0].at[:, pl.ds(0, 128)], win_v.at[0], sem).wait()

    w = w_r[...]
    tails_all = tails_r[...]
    res = b_r[...]
    lane = lax.broadcasted_iota(jnp.int32, (EMB, 128), 1)
    for t in range(4):
        acc = jnp.zeros((EMB, 128), jnp.float32)
        tlane = lax.broadcasted_iota(jnp.int32, (EMB, TWS[t]), 1)
        tacc = jnp.zeros((EMB, TWS[t]), jnp.float32)
        tail = tails_all[:, _TOFF[t]:_TOFF[t + 1]]
        for j in range(SEQ):
            a = idx_r[t, j]
            wb = jnp.minimum((a // 128) * 128, THRS[t] - 128)
            acc = acc + jnp.where(lane == a - wb, win_v[t * SEQ + j], 0.0)
            tacc = tacc + jnp.where(tlane == a - THRS[t], tail, 0.0)
        pooled = (acc.sum(axis=1, keepdims=True)
                  + tacc.sum(axis=1, keepdims=True)) * fuse_r[t]
        res = res + jnp.dot(w[:, t * EMB:(t + 1) * EMB], pooled,
                            preferred_element_type=jnp.float32)
    res_o[...] = res
    p_o[...] = 1.0 / (1.0 + jnp.exp(-res))


_tc_fwd = pl.pallas_call(
    _tc_body,
    out_shape=(jax.ShapeDtypeStruct((V3, 1), jnp.float32),
               jax.ShapeDtypeStruct((V3, 1), jnp.float32)),
    in_specs=[pl.BlockSpec(memory_space=pltpu.SMEM)]
    + [pl.BlockSpec(memory_space=pl.ANY)] * 4
    + [pl.BlockSpec(memory_space=pltpu.VMEM)]
    + [pl.BlockSpec(memory_space=pltpu.SMEM)]
    + [pl.BlockSpec(memory_space=pltpu.VMEM)] * 2,
    out_specs=(pl.BlockSpec(memory_space=pltpu.VMEM),
               pl.BlockSpec(memory_space=pltpu.VMEM)),
    scratch_shapes=[pltpu.VMEM((4 * SEQ, EMB, 128), jnp.float32),
                    pltpu.SemaphoreType.DMA],
    compiler_params=pltpu.CompilerParams(
        vmem_limit_bytes=48 * 1024 * 1024),
)


_NFC = 62


@functools.partial(
    pl.kernel,
    out_type=jax.ShapeDtypeStruct((NW, 1, 128), jnp.float32),
    mesh=_mesh,
    compiler_params=_sc_params,
    scratch_types=[
        pltpu.VMEM((1, V3), jnp.float32),
        pltpu.VMEM((RPW, V3), jnp.float32),
        pltpu.VMEM((1, 128), jnp.float32),
    ],
)
def _ddi_kernel(p_h, a_h, out_h, p_v, a_v, tot_v):
    wid = lax.axis_index("s") * NC + lax.axis_index("c")
    r0 = jnp.minimum(wid * RPW, V3 - RPW)
    zero = jnp.zeros((LANES,), jnp.float32)
    pltpu.sync_copy(p_h, p_v)
    pltpu.sync_copy(a_h.at[pl.ds(r0, RPW), :], a_v)
    tailm = (lax.iota(jnp.int32, LANES) >= 8).astype(jnp.float32)

    def row(r, total):
        def dot(k, acc):
            return acc + a_v[r, pl.ds(16 * k, LANES)] * p_v[0, pl.ds(16 * k, LANES)]

        rowacc = lax.fori_loop(0, _NFC, dot, zero)
        rowacc = rowacc + (a_v[r, pl.ds(984, LANES)] * tailm
                           * p_v[0, pl.ds(984, LANES)])
        g = r0 + r
        coef = _splat(p_v.at[0], g)
        valid = (jnp.broadcast_to(g, (LANES,)) >= wid * RPW).astype(jnp.float32)
        return total + coef * valid * rowacc

    for m in range(1, 8):
        tot_v[0, pl.ds(16 * m, LANES)] = zero
    tot_v[0, pl.ds(0, LANES)] = lax.fori_loop(0, RPW, row, zero)
    pltpu.sync_copy(tot_v, out_h.at[wid])


def kernel(diag_idx, proc_idx, sym_idx, med_idx, emb0, emb1, emb2, emb3,
           fuse_w, W_q, b_q, ddi_adj):
    idx = jnp.stack([i.astype(jnp.int32)
                     for i in (diag_idx, proc_idx, sym_idx, med_idx)])
    embs = (emb0, emb1, emb2, emb3)
    embsT = [e.T for e in embs]
    tails_cat = jnp.concatenate(
        [e[t:].T for e, t in zip(embs, THRS)], axis=1)
    res_c, p_c = _tc_fwd(idx, *embsT, tails_cat, fuse_w.reshape(4),
                         W_q.T, b_q[:, None])
    res = res_c.reshape(1, V3)
    p = p_c.reshape(1, V3)
    partials = _ddi_kernel(p, ddi_adj)
    return res, 0.0005 * jnp.sum(partials)


# speedup vs baseline: 4.0061x; 1.1274x over previous
import functools

import jax
import jax.numpy as jnp
from jax import lax
from jax.experimental import pallas as pl
from jax.experimental.pallas import tpu as pltpu
from jax.experimental.pallas import tpu_sc as plsc

NC, NS, LANES = 2, 16, 16
NW = NC * NS
EMB = 64
SEQ = 50
V3 = 1000
RPW = 32
VOCABS = (100000, 100000, 100000, 1000)
THRS = tuple((v // 128) * 128 for v in VOCABS)
TWS = tuple(v - t for v, t in zip(VOCABS, THRS))

_mesh = plsc.VectorSubcoreMesh(core_axis_name="c", subcore_axis_name="s",
                               num_cores=NC, num_subcores=NS)
_sc_params = pltpu.CompilerParams(needs_layout_passes=False,
                                  use_tc_tiling_on_sc=True)


def _splat(ref, j):
    return plsc.load_gather(ref, [jnp.broadcast_to(j, (LANES,)).astype(jnp.int32)])


_TOFF = (0, 32, 64, 96, 200)


def _tc_body(idx_r, e0, e1, e2, e3, tails_r,
             fuse_r, w_r, b_r, res_o, p_o, win_v, sem):
    tabs = (e0, e1, e2, e3)
    for t in range(4):
        for j in range(SEQ):
            a = idx_r[t, j]
            wb = pl.multiple_of(
                jnp.minimum((a // 128) * 128, THRS[t] - 128), 128)
            pltpu.make_async_copy(
                tabs[t].at[:, pl.ds(wb, 128)],
                win_v.at[t * SEQ + j], sem).start()
    for n in range(4 * SEQ):
        pltpu.make_async_copy(
            tabs[0].at[:, pl.ds(0, 128)], win_v.at[0], sem).wait()

    w = w_r[...]
    tails_all = tails_r[...]
    res = b_r[...]
    lane = lax.broadcasted_iota(jnp.int32, (EMB, 128), 1)
    for t in range(4):
        acc = jnp.zeros((EMB, 128), jnp.float32)
        tlane = lax.broadcasted_iota(jnp.int32, (EMB, TWS[t]), 1)
        tacc = jnp.zeros((EMB, TWS[t]), jnp.float32)
        tail = tails_all[:, _TOFF[t]:_TOFF[t + 1]]
        for j in range(SEQ):
            a = idx_r[t, j]
            wb = jnp.minimum((a // 128) * 128, THRS[t] - 128)
            acc = acc + jnp.where(lane == a - wb, win_v[t * SEQ + j], 0.0)
            tacc = tacc + jnp.where(tlane == a - THRS[t], tail, 0.0)
        pooled = (acc.sum(axis=1, keepdims=True)
                  + tacc.sum(axis=1, keepdims=True)) * fuse_r[t]
        res = res + jnp.dot(w[:, t * EMB:(t + 1) * EMB], pooled,
                            preferred_element_type=jnp.float32)
    res_o[...] = res
    p_o[...] = 1.0 / (1.0 + jnp.exp(-res))


_tc_fwd = pl.pallas_call(
    _tc_body,
    out_shape=(jax.ShapeDtypeStruct((V3, 1), jnp.float32),
               jax.ShapeDtypeStruct((V3, 1), jnp.float32)),
    in_specs=[pl.BlockSpec(memory_space=pltpu.SMEM)]
    + [pl.BlockSpec(memory_space=pl.ANY)] * 4
    + [pl.BlockSpec(memory_space=pltpu.VMEM)]
    + [pl.BlockSpec(memory_space=pltpu.SMEM)]
    + [pl.BlockSpec(memory_space=pltpu.VMEM)] * 2,
    out_specs=(pl.BlockSpec(memory_space=pltpu.VMEM),
               pl.BlockSpec(memory_space=pltpu.VMEM)),
    scratch_shapes=[pltpu.VMEM((4 * SEQ, EMB, 128), jnp.float32),
                    pltpu.SemaphoreType.DMA],
    compiler_params=pltpu.CompilerParams(
        vmem_limit_bytes=48 * 1024 * 1024),
)


_NFC = 62


@functools.partial(
    pl.kernel,
    out_type=jax.ShapeDtypeStruct((NW, 1, 128), jnp.float32),
    mesh=_mesh,
    compiler_params=_sc_params,
    scratch_types=[
        pltpu.VMEM((1, V3), jnp.float32),
        pltpu.VMEM((RPW, V3), jnp.float32),
        pltpu.VMEM((1, 128), jnp.float32),
    ],
)
def _ddi_kernel(p_h, a_h, out_h, p_v, a_v, tot_v):
    wid = lax.axis_index("s") * NC + lax.axis_index("c")
    r0 = jnp.minimum(wid * RPW, V3 - RPW)
    zero = jnp.zeros((LANES,), jnp.float32)
    pltpu.sync_copy(p_h, p_v)
    pltpu.sync_copy(a_h.at[pl.ds(r0, RPW), :], a_v)
    tailm = (lax.iota(jnp.int32, LANES) >= 8).astype(jnp.float32)
    ptail = p_v[0, pl.ds(984, LANES)] * tailm

    total = zero
    for g in range(4):
        def dot(k, accs):
            pk = p_v[0, pl.ds(16 * k, LANES)]
            return tuple(a + a_v[g * 8 + i, pl.ds(16 * k, LANES)] * pk
                         for i, a in enumerate(accs))

        accs = lax.fori_loop(0, _NFC, dot, (zero,) * 8)
        for i in range(8):
            r = g * 8 + i
            rowacc = accs[i] + a_v[r, pl.ds(984, LANES)] * ptail
            gg = r0 + r
            coef = _splat(p_v.at[0], gg)
            valid = (jnp.broadcast_to(gg, (LANES,)) >= wid * RPW).astype(jnp.float32)
            total = total + coef * valid * rowacc

    for m in range(1, 8):
        tot_v[0, pl.ds(16 * m, LANES)] = zero
    tot_v[0, pl.ds(0, LANES)] = total
    pltpu.sync_copy(tot_v, out_h.at[wid])


def kernel(diag_idx, proc_idx, sym_idx, med_idx, emb0, emb1, emb2, emb3,
           fuse_w, W_q, b_q, ddi_adj):
    idx = jnp.stack([i.astype(jnp.int32)
                     for i in (diag_idx, proc_idx, sym_idx, med_idx)])
    embs = (emb0, emb1, emb2, emb3)
    embsT = [e.T for e in embs]
    tails_cat = jnp.concatenate(
        [e[t:].T for e, t in zip(embs, THRS)], axis=1)
    res_c, p_c = _tc_fwd(idx, *embsT, tails_cat, fuse_w.reshape(4),
                         W_q.T, b_q[:, None])
    res = res_c.reshape(1, V3)
    p = p_c.reshape(1, V3)
    partials = _ddi_kernel(p, ddi_adj)
    return res, 0.0005 * jnp.sum(partials)


# speedup vs baseline: 4.0413x; 1.0088x over previous
import functools

import jax
import jax.numpy as jnp
from jax import lax
from jax.experimental import pallas as pl
from jax.experimental.pallas import tpu as pltpu
from jax.experimental.pallas import tpu_sc as plsc

NC, NS, LANES = 2, 16, 16
NW = NC * NS
EMB = 64
SEQ = 50
V3 = 1000
RPW = 32
VOCABS = (100000, 100000, 100000, 1000)
THRS = tuple((v // 128) * 128 for v in VOCABS)
TWS = tuple(v - t for v, t in zip(VOCABS, THRS))

_mesh = plsc.VectorSubcoreMesh(core_axis_name="c", subcore_axis_name="s",
                               num_cores=NC, num_subcores=NS)
_sc_params = pltpu.CompilerParams(needs_layout_passes=False,
                                  use_tc_tiling_on_sc=True)


def _splat(ref, j):
    return plsc.load_gather(ref, [jnp.broadcast_to(j, (LANES,)).astype(jnp.int32)])


_TOFF = (0, 32, 64, 96, 200)


def _tc_body(idx_r, e0, e1, e2, e3, tails_r,
             fuse_r, w_r, b_r, res_o, p_o, win_v, sem):
    tabs = (e0, e1, e2, e3)
    for t in range(4):
        for j in range(SEQ):
            a = idx_r[t, j]
            wb = pl.multiple_of(
                jnp.minimum((a // 128) * 128, THRS[t] - 128), 128)
            pltpu.make_async_copy(
                tabs[t].at[:, pl.ds(wb, 128)],
                win_v.at[t * SEQ + j], sem).start()
    for n in range(4 * SEQ):
        pltpu.make_async_copy(
            tabs[0].at[:, pl.ds(0, 128)], win_v.at[0], sem).wait()

    w = w_r[...]
    tails_all = tails_r[...]
    res = b_r[...]
    lane = lax.broadcasted_iota(jnp.int32, (EMB, 128), 1)
    for t in range(4):
        acc = jnp.zeros((EMB, 128), jnp.float32)
        tlane = lax.broadcasted_iota(jnp.int32, (EMB, TWS[t]), 1)
        tacc = jnp.zeros((EMB, TWS[t]), jnp.float32)
        tail = tails_all[:, _TOFF[t]:_TOFF[t + 1]]
        for j in range(SEQ):
            a = idx_r[t, j]
            wb = jnp.minimum((a // 128) * 128, THRS[t] - 128)
            acc = acc + jnp.where(lane == a - wb, win_v[t * SEQ + j], 0.0)
            tacc = tacc + jnp.where(tlane == a - THRS[t], tail, 0.0)
        pooled = (acc.sum(axis=1, keepdims=True)
                  + tacc.sum(axis=1, keepdims=True)) * fuse_r[t]
        res = res + jnp.dot(w[:, t * EMB:(t + 1) * EMB], pooled,
                            preferred_element_type=jnp.float32)
    res_o[...] = res
    p_o[...] = 1.0 / (1.0 + jnp.exp(-res))


_tc_fwd = pl.pallas_call(
    _tc_body,
    out_shape=(jax.ShapeDtypeStruct((V3, 1), jnp.float32),
               jax.ShapeDtypeStruct((V3, 1), jnp.float32)),
    in_specs=[pl.BlockSpec(memory_space=pltpu.SMEM)]
    + [pl.BlockSpec(memory_space=pl.ANY)] * 4
    + [pl.BlockSpec(memory_space=pltpu.VMEM)]
    + [pl.BlockSpec(memory_space=pltpu.SMEM)]
    + [pl.BlockSpec(memory_space=pltpu.VMEM)] * 2,
    out_specs=(pl.BlockSpec(memory_space=pltpu.VMEM),
               pl.BlockSpec(memory_space=pltpu.VMEM)),
    scratch_shapes=[pltpu.VMEM((4 * SEQ, EMB, 128), jnp.float32),
                    pltpu.SemaphoreType.DMA],
    compiler_params=pltpu.CompilerParams(
        vmem_limit_bytes=48 * 1024 * 1024),
)


_NFC = 62


@functools.partial(
    pl.kernel,
    out_type=jax.ShapeDtypeStruct((NW, 1, 128), jnp.float32),
    mesh=_mesh,
    compiler_params=_sc_params,
    scratch_types=[
        pltpu.VMEM((1, V3), jnp.float32),
        pltpu.VMEM((RPW, V3), jnp.float32),
        pltpu.VMEM((1, 128), jnp.float32),
    ],
)
def _ddi_kernel(p_h, a_h, out_h, p_v, a_v, tot_v):
    wid = lax.axis_index("s") * NC + lax.axis_index("c")
    r0 = jnp.minimum(wid * RPW, V3 - RPW)
    zero = jnp.zeros((LANES,), jnp.float32)
    pltpu.sync_copy(p_h, p_v)
    pltpu.sync_copy(a_h.at[pl.ds(r0, RPW), :], a_v)
    tailm = (lax.iota(jnp.int32, LANES) >= 8).astype(jnp.float32)
    ptail = p_v[0, pl.ds(984, LANES)] * tailm

    total = zero
    for g in range(2):
        def dot(k, accs):
            pk = p_v[0, pl.ds(16 * k, LANES)]
            return tuple(a + a_v[g * 16 + i, pl.ds(16 * k, LANES)] * pk
                         for i, a in enumerate(accs))

        accs = lax.fori_loop(0, _NFC, dot, (zero,) * 16)
        for i in range(16):
            r = g * 16 + i
            rowacc = accs[i] + a_v[r, pl.ds(984, LANES)] * ptail
            gg = r0 + r
            coef = _splat(p_v.at[0], gg)
            valid = (jnp.broadcast_to(gg, (LANES,)) >= wid * RPW).astype(jnp.float32)
            total = total + coef * valid * rowacc

    for m in range(1, 8):
        tot_v[0, pl.ds(16 * m, LANES)] = zero
    tot_v[0, pl.ds(0, LANES)] = total
    pltpu.sync_copy(tot_v, out_h.at[wid])


def kernel(diag_idx, proc_idx, sym_idx, med_idx, emb0, emb1, emb2, emb3,
           fuse_w, W_q, b_q, ddi_adj):
    idx = jnp.stack([i.astype(jnp.int32)
                     for i in (diag_idx, proc_idx, sym_idx, med_idx)])
    embs = (emb0, emb1, emb2, emb3)
    embsT = [e.T for e in embs]
    tails_cat = jnp.concatenate(
        [e[t:].T for e, t in zip(embs, THRS)], axis=1)
    res_c, p_c = _tc_fwd(idx, *embsT, tails_cat, fuse_w.reshape(4),
                         W_q.T, b_q[:, None])
    res = res_c.reshape(1, V3)
    p = p_c.reshape(1, V3)
    partials = _ddi_kernel(p, ddi_adj)
    return res, 0.0005 * jnp.sum(partials)


# speedup vs baseline: 4.0770x; 1.0088x over previous
import functools

import jax
import jax.numpy as jnp
from jax import lax
from jax.experimental import pallas as pl
from jax.experimental.pallas import tpu as pltpu
from jax.experimental.pallas import tpu_sc as plsc

NC, NS, LANES = 2, 16, 16
NW = NC * NS
EMB = 64
SEQ = 50
V3 = 1000
RPW = 32
VOCABS = (100000, 100000, 100000, 1000)
THRS = tuple((v // 128) * 128 for v in VOCABS)
TWS = tuple(v - t for v, t in zip(VOCABS, THRS))

_mesh = plsc.VectorSubcoreMesh(core_axis_name="c", subcore_axis_name="s",
                               num_cores=NC, num_subcores=NS)
_sc_params = pltpu.CompilerParams(needs_layout_passes=False,
                                  use_tc_tiling_on_sc=True)


def _splat(ref, j):
    return plsc.load_gather(ref, [jnp.broadcast_to(j, (LANES,)).astype(jnp.int32)])


_TOFF = (0, 32, 64, 96, 200)


def _tc_body(idx_r, e0, e1, e2, e3, tails_r,
             fuse_r, w_r, b_r, res_o, p_o, win_v, sem):
    tabs = (e0, e1, e2, e3)
    for t in range(4):
        for j in range(SEQ):
            a = idx_r[t, j]
            wb = pl.multiple_of(
                jnp.minimum((a // 128) * 128, THRS[t] - 128), 128)
            pltpu.make_async_copy(
                tabs[t].at[:, pl.ds(wb, 128)],
                win_v.at[t * SEQ + j], sem).start()
    for n in range(4 * SEQ):
        pltpu.make_async_copy(
            tabs[0].at[:, pl.ds(0, 128)], win_v.at[0], sem).wait()

    w = w_r[...]
    tails_all = tails_r[...]
    res = b_r[...]
    lane = lax.broadcasted_iota(jnp.int32, (EMB, 128), 1)
    for t in range(4):
        acc = jnp.zeros((EMB, 128), jnp.float32)
        tlane = lax.broadcasted_iota(jnp.int32, (EMB, TWS[t]), 1)
        tacc = jnp.zeros((EMB, TWS[t]), jnp.float32)
        tail = tails_all[:, _TOFF[t]:_TOFF[t + 1]]
        for j in range(SEQ):
            a = idx_r[t, j]
            wb = jnp.minimum((a // 128) * 128, THRS[t] - 128)
            acc = acc + jnp.where(lane == a - wb, win_v[t * SEQ + j], 0.0)
            tacc = tacc + jnp.where(tlane == a - THRS[t], tail, 0.0)
        pooled = (acc.sum(axis=1, keepdims=True)
                  + tacc.sum(axis=1, keepdims=True)) * fuse_r[t]
        res = res + lax.dot_general(
            pooled, w[:, t * EMB:(t + 1) * EMB],
            (((0,), (1,)), ((), ())),
            preferred_element_type=jnp.float32)
    res_o[...] = res
    p_o[...] = 1.0 / (1.0 + jnp.exp(-res))


_tc_fwd = pl.pallas_call(
    _tc_body,
    out_shape=(jax.ShapeDtypeStruct((1, V3), jnp.float32),
               jax.ShapeDtypeStruct((1, V3), jnp.float32)),
    in_specs=[pl.BlockSpec(memory_space=pltpu.SMEM)]
    + [pl.BlockSpec(memory_space=pl.ANY)] * 4
    + [pl.BlockSpec(memory_space=pltpu.VMEM)]
    + [pl.BlockSpec(memory_space=pltpu.SMEM)]
    + [pl.BlockSpec(memory_space=pltpu.VMEM)] * 2,
    out_specs=(pl.BlockSpec(memory_space=pltpu.VMEM),
               pl.BlockSpec(memory_space=pltpu.VMEM)),
    scratch_shapes=[pltpu.VMEM((4 * SEQ, EMB, 128), jnp.float32),
                    pltpu.SemaphoreType.DMA],
    compiler_params=pltpu.CompilerParams(
        vmem_limit_bytes=48 * 1024 * 1024),
)


_NFC = 62


@functools.partial(
    pl.kernel,
    out_type=jax.ShapeDtypeStruct((NW, 1, 128), jnp.float32),
    mesh=_mesh,
    compiler_params=_sc_params,
    scratch_types=[
        pltpu.VMEM((1, V3), jnp.float32),
        pltpu.VMEM((RPW, V3), jnp.float32),
        pltpu.VMEM((1, 128), jnp.float32),
    ],
)
def _ddi_kernel(p_h, a_h, out_h, p_v, a_v, tot_v):
    wid = lax.axis_index("s") * NC + lax.axis_index("c")
    r0 = jnp.minimum(wid * RPW, V3 - RPW)
    zero = jnp.zeros((LANES,), jnp.float32)
    pltpu.sync_copy(p_h, p_v)
    pltpu.sync_copy(a_h.at[pl.ds(r0, RPW), :], a_v)
    tailm = (lax.iota(jnp.int32, LANES) >= 8).astype(jnp.float32)
    ptail = p_v[0, pl.ds(984, LANES)] * tailm

    total = zero
    for g in range(2):
        def dot(k, accs):
            pk = p_v[0, pl.ds(16 * k, LANES)]
            return tuple(a + a_v[g * 16 + i, pl.ds(16 * k, LANES)] * pk
                         for i, a in enumerate(accs))

        accs = lax.fori_loop(0, _NFC, dot, (zero,) * 16)
        for i in range(16):
            r = g * 16 + i
            rowacc = accs[i] + a_v[r, pl.ds(984, LANES)] * ptail
            gg = r0 + r
            coef = _splat(p_v.at[0], gg)
            valid = (jnp.broadcast_to(gg, (LANES,)) >= wid * RPW).astype(jnp.float32)
            total = total + coef * valid * rowacc

    for m in range(1, 8):
        tot_v[0, pl.ds(16 * m, LANES)] = zero
    tot_v[0, pl.ds(0, LANES)] = total
    pltpu.sync_copy(tot_v, out_h.at[wid])


def kernel(diag_idx, proc_idx, sym_idx, med_idx, emb0, emb1, emb2, emb3,
           fuse_w, W_q, b_q, ddi_adj):
    idx = jnp.stack([i.astype(jnp.int32)
                     for i in (diag_idx, proc_idx, sym_idx, med_idx)])
    embs = (emb0, emb1, emb2, emb3)
    embsT = [e.T for e in embs]
    tails_cat = jnp.concatenate(
        [e[t:].T for e, t in zip(embs, THRS)], axis=1)
    res, p = _tc_fwd(idx, *embsT, tails_cat, fuse_w.reshape(4),
                     W_q.T, b_q[None, :])
    partials = _ddi_kernel(p, ddi_adj)
    return res, 0.0005 * jnp.sum(partials)


# speedup vs baseline: 4.1663x; 1.0219x over previous
import functools

import jax
import jax.numpy as jnp
from jax import lax
from jax.experimental import pallas as pl
from jax.experimental.pallas import tpu as pltpu
from jax.experimental.pallas import tpu_sc as plsc

NC, NS, LANES = 2, 16, 16
NW = NC * NS
EMB = 64
SEQ = 50
V3 = 1000
RPW = 32
VOCABS = (100000, 100000, 100000, 1000)
THRS = tuple((v // 128) * 128 for v in VOCABS)
TWS = tuple(v - t for v, t in zip(VOCABS, THRS))

_mesh = plsc.VectorSubcoreMesh(core_axis_name="c", subcore_axis_name="s",
                               num_cores=NC, num_subcores=NS)
_sc_params = pltpu.CompilerParams(needs_layout_passes=False,
                                  use_tc_tiling_on_sc=True)


def _splat(ref, j):
    return plsc.load_gather(ref, [jnp.broadcast_to(j, (LANES,)).astype(jnp.int32)])


_TOFF = (0, 32, 64, 96, 200)


def _tc_body(idx_r, e0, e1, e2, e3, tails_r,
             fuse_r, w_r, b_r, res_o, p_o, win_v, sem):
    tabs = (e0, e1, e2, e3)
    for t in range(4):
        for j in range(SEQ):
            a = idx_r[t, j]
            wb = pl.multiple_of(
                jnp.minimum((a // 128) * 128, THRS[t] - 128), 128)
            pltpu.make_async_copy(
                tabs[t].at[:, pl.ds(wb, 128)],
                win_v.at[t * SEQ + j], sem).start()
    for n in range(4 * SEQ):
        pltpu.make_async_copy(
            tabs[0].at[:, pl.ds(0, 128)], win_v.at[0], sem).wait()

    w = w_r[...]
    tails_all = tails_r[...]
    res = b_r[...]
    lane = lax.broadcasted_iota(jnp.int32, (EMB, 128), 1)
    for t in range(4):
        acc = jnp.zeros((EMB, 128), jnp.float32)
        tlane = lax.broadcasted_iota(jnp.int32, (EMB, TWS[t]), 1)
        tacc = jnp.zeros((EMB, TWS[t]), jnp.float32)
        tail = tails_all[:, _TOFF[t]:_TOFF[t + 1]]
        for j in range(SEQ):
            a = idx_r[t, j]
            wb = jnp.minimum((a // 128) * 128, THRS[t] - 128)
            acc = acc + jnp.where(lane == a - wb, win_v[t * SEQ + j], 0.0)
            tacc = tacc + jnp.where(tlane == a - THRS[t], tail, 0.0)
        pooled = (acc.sum(axis=1, keepdims=True)
                  + tacc.sum(axis=1, keepdims=True)) * fuse_r[t]
        res = res + lax.dot_general(
            pooled, w[:, t * EMB:(t + 1) * EMB],
            (((0,), (1,)), ((), ())),
            preferred_element_type=jnp.float32)
    res_o[...] = res
    p_o[...] = 1.0 / (1.0 + jnp.exp(-res))


_tc_fwd = pl.pallas_call(
    _tc_body,
    out_shape=(jax.ShapeDtypeStruct((1, V3), jnp.float32),
               jax.ShapeDtypeStruct((1, V3), jnp.float32)),
    in_specs=[pl.BlockSpec(memory_space=pltpu.SMEM)]
    + [pl.BlockSpec(memory_space=pl.ANY)] * 4
    + [pl.BlockSpec(memory_space=pltpu.VMEM)]
    + [pl.BlockSpec(memory_space=pltpu.SMEM)]
    + [pl.BlockSpec(memory_space=pltpu.VMEM)] * 2,
    out_specs=(pl.BlockSpec(memory_space=pltpu.VMEM),
               pl.BlockSpec(memory_space=pltpu.VMEM)),
    scratch_shapes=[pltpu.VMEM((4 * SEQ, EMB, 128), jnp.float32),
                    pltpu.SemaphoreType.DMA],
    compiler_params=pltpu.CompilerParams(
        vmem_limit_bytes=48 * 1024 * 1024),
)


_NFC = 62


@functools.partial(
    pl.kernel,
    out_type=jax.ShapeDtypeStruct((NW, 1, 128), jnp.float32),
    mesh=_mesh,
    compiler_params=_sc_params,
    scratch_types=[
        pltpu.VMEM((1, V3), jnp.float32),
        pltpu.VMEM((RPW, V3), jnp.float32),
        pltpu.VMEM((1, 128), jnp.float32),
        pltpu.SemaphoreType.DMA,
        pltpu.SemaphoreType.DMA,
    ],
)
def _ddi_kernel(p_h, a_h, out_h, p_v, a_v, tot_v, sem0, sem1):
    wid = lax.axis_index("s") * NC + lax.axis_index("c")
    r0 = jnp.minimum(wid * RPW, V3 - RPW)
    zero = jnp.zeros((LANES,), jnp.float32)
    half = RPW // 2
    cps = [pltpu.async_copy(a_h.at[pl.ds(r0 + g * half, half), :],
                            a_v.at[pl.ds(g * half, half), :], s)
           for g, s in ((0, sem0), (1, sem1))]
    pltpu.sync_copy(p_h, p_v)
    tailm = (lax.iota(jnp.int32, LANES) >= 8).astype(jnp.float32)
    ptail = p_v[0, pl.ds(984, LANES)] * tailm

    total = zero
    for g in range(2):
        cps[g].wait()

        def dot(k, accs):
            pk = p_v[0, pl.ds(16 * k, LANES)]
            return tuple(a + a_v[g * 16 + i, pl.ds(16 * k, LANES)] * pk
                         for i, a in enumerate(accs))

        accs = lax.fori_loop(0, _NFC, dot, (zero,) * 16)
        for i in range(16):
            r = g * 16 + i
            rowacc = accs[i] + a_v[r, pl.ds(984, LANES)] * ptail
            gg = r0 + r
            coef = _splat(p_v.at[0], gg)
            valid = (jnp.broadcast_to(gg, (LANES,)) >= wid * RPW).astype(jnp.float32)
            total = total + coef * valid * rowacc

    for m in range(1, 8):
        tot_v[0, pl.ds(16 * m, LANES)] = zero
    tot_v[0, pl.ds(0, LANES)] = total
    pltpu.sync_copy(tot_v, out_h.at[wid])


def kernel(diag_idx, proc_idx, sym_idx, med_idx, emb0, emb1, emb2, emb3,
           fuse_w, W_q, b_q, ddi_adj):
    idx = jnp.stack([i.astype(jnp.int32)
                     for i in (diag_idx, proc_idx, sym_idx, med_idx)])
    embs = (emb0, emb1, emb2, emb3)
    embsT = [e.T for e in embs]
    tails_cat = jnp.concatenate(
        [e[t:].T for e, t in zip(embs, THRS)], axis=1)
    res, p = _tc_fwd(idx, *embsT, tails_cat, fuse_w.reshape(4),
                     W_q.T, b_q[None, :])
    partials = _ddi_kernel(p, ddi_adj)
    return res, 0.0005 * jnp.sum(partials)
